# baseline (speedup 1.0000x reference)
import functools

import jax
import jax.numpy as jnp
from jax import lax
from jax.experimental import pallas as pl
from jax.experimental.pallas import tpu as pltpu
from jax.experimental.pallas import tpu_sc as plsc

N = 10000
E = 320000
D = 128
G = 64
C = 16

NC = 2
NS = 16
NW = NC * NS
CHUNK = 64
NCHUNK = 160
W = 16
NWIN = NCHUNK // W
SPW = W // 4
EPW = CHUNK * NCHUNK
E_PAD = NW * EPW
N_PAD = 10240
RPS = N_PAD // NS

DCH = 128
DNC = EPW // DCH


@functools.lru_cache(maxsize=None)
def _make_deg_kernel():
    mesh = plsc.VectorSubcoreMesh(core_axis_name="c", subcore_axis_name="s")
    return functools.partial(
        pl.kernel, mesh=mesh,
        out_type=jax.ShapeDtypeStruct((NC, N_PAD), jnp.float32),
        scratch_types=[
            pltpu.VMEM((DNC, DCH), jnp.int32),
            pltpu.VMEM((DNC, DCH), jnp.float32),
            pltpu.VMEM((RPS,), jnp.float32),
            pltpu.SemaphoreType.DMA,
            pltpu.SemaphoreType.DMA,
            pltpu.SemaphoreType.DMA,
            pltpu.SemaphoreType.DMA,
            pltpu.VMEM_SHARED((N_PAD,), jnp.float32),
        ],
    )(_deg_body)


def _deg_body(dst_hbm, ew_hbm, out_hbm, dst_v, ew_v, zbuf, ds0, ds1, ds2, ds3, sdeg):
    c = lax.axis_index("c")
    s = lax.axis_index("s")
    wid = s * NC + c
    pltpu.sync_copy(dst_hbm.at[wid], dst_v)
    pltpu.sync_copy(ew_hbm.at[wid], ew_v)

    def zb(k, carry):
        zbuf[pl.ds(k * 16, 16)] = jnp.zeros((16,), jnp.float32)
        return carry
    lax.fori_loop(0, RPS // 16, zb, None)
    pltpu.sync_copy(zbuf, sdeg.at[pl.ds(s * RPS, RPS)])
    plsc.subcore_barrier()

    dsem = [ds0, ds1, ds2, ds3]

    def chunk4(jj, carry):
        for b in range(4):
            j = jj * 4 + b
            pltpu.async_copy(ew_v.at[j], sdeg.at[dst_v.at[j]], dsem[b], add=True)

            @pl.when(j >= 4)
            def _drain():
                pltpu.make_async_copy(
                    ew_v.at[j - 4], sdeg.at[dst_v.at[j - 4]], dsem[b]).wait()
        return carry
    lax.fori_loop(0, DNC // 4, chunk4, None)
    for b in range(4):
        pltpu.make_async_copy(
            ew_v.at[DNC - 4 + b], sdeg.at[dst_v.at[DNC - 4 + b]],
            dsem[b]).wait()
    plsc.subcore_barrier()
    pltpu.sync_copy(sdeg.at[pl.ds(s * RPS, RPS)], out_hbm.at[c, pl.ds(s * RPS, RPS)])


@functools.lru_cache(maxsize=None)
def _make_edge_kernel():
    mesh = plsc.VectorSubcoreMesh(core_axis_name="c", subcore_axis_name="s")
    return functools.partial(
        pl.kernel, mesh=mesh,
        out_type=jax.ShapeDtypeStruct((NC, N_PAD, D), jnp.float32),
        scratch_types=[
            pltpu.VMEM((2, W, CHUNK), jnp.int32),
            pltpu.VMEM((2, W, CHUNK), jnp.int32),
            pltpu.VMEM((2, W, CHUNK), jnp.float32),
            pltpu.VMEM((4, CHUNK, D), jnp.float32),
            pltpu.SemaphoreType.DMA,
            pltpu.SemaphoreType.DMA,
            pltpu.SemaphoreType.DMA,
            pltpu.SemaphoreType.DMA,
            pltpu.SemaphoreType.DMA,
            pltpu.SemaphoreType.DMA,
            pltpu.SemaphoreType.DMA,
            pltpu.SemaphoreType.DMA,
            pltpu.SemaphoreType.DMA,
            pltpu.SemaphoreType.DMA,
            pltpu.VMEM_SHARED((N_PAD, D), jnp.float32),
        ],
    )(_edge_body)


def _edge_body(xs_hbm, src_hbm, dst_hbm, ew_hbm, out_hbm,
               win_src, win_dst, win_ew, rows4,
               gs0, gs1, gs2, gs3, ss0, ss1, ss2, ss3, ws0, ws1, acc):
    c = lax.axis_index("c")
    s = lax.axis_index("s")
    wid = s * NC + c
    gsem = [gs0, gs1, gs2, gs3]
    ssem = [ss0, ss1, ss2, ss3]
    wsem = [ws0, ws1]

    def win_ops(w, wb):
        return [
            (src_hbm.at[wid, pl.ds(w * W, W)], win_src.at[wb]),
            (dst_hbm.at[wid, pl.ds(w * W, W)], win_dst.at[wb]),
            (ew_hbm.at[wid, pl.ds(w * W, W)], win_ew.at[wb]),
        ]

    def win_load_start(w, wb):
        for sr, ds_ in win_ops(w, wb):
            pltpu.async_copy(sr, ds_, wsem[wb])

    def win_load_wait(w, wb):
        for sr, ds_ in win_ops(w, wb):
            pltpu.make_async_copy(sr, ds_, wsem[wb]).wait()

    def zr(i, carry):
        for h in range(D // 16):
            rows4[0, i, pl.ds(h * 16, 16)] = jnp.zeros((16,), jnp.float32)
        return carry
    lax.fori_loop(0, CHUNK, zr, None)
    for t in range(RPS // CHUNK):
        pltpu.async_copy(rows4.at[0], acc.at[pl.ds(s * RPS + t * CHUNK, CHUNK)],
                         gs0)
    for t in range(RPS // CHUNK):
        pltpu.make_async_copy(rows4.at[0],
                              acc.at[pl.ds(s * RPS + t * CHUNK, CHUNK)],
                              gs0).wait()
    plsc.subcore_barrier()

    def gather_start(j, b):
        pltpu.async_copy(xs_hbm.at[win_src.at[(j // W) % 2, j % W]],
                         rows4.at[b], gsem[b])

    def gather_wait(j, b):
        pltpu.make_async_copy(xs_hbm.at[win_src.at[(j // W) % 2, j % W]],
                              rows4.at[b], gsem[b]).wait()

    def scatter_start(j, b):
        pltpu.async_copy(rows4.at[b], acc.at[win_dst.at[(j // W) % 2, j % W]],
                         ssem[b], add=True)

    def scatter_wait(j, b):
        pltpu.make_async_copy(rows4.at[b], acc.at[win_dst.at[(j // W) % 2, j % W]],
                              ssem[b]).wait()

    def scale(j, b):
        wb = (j // W) % 2
        row = j % W

        def group(g2, carry):
            for u in range(2):
                g = g2 * 2 + u
                ew16 = win_ew[wb, row, pl.ds(g * 16, 16)]
                for k in range(16):
                    ewb = jnp.full((16,), ew16[k])
                    i = g * 16 + k
                    for h in range(D // 16):
                        sl = rows4[b, i, pl.ds(h * 16, 16)]
                        rows4[b, i, pl.ds(h * 16, 16)] = sl * ewb
            return carry
        lax.fori_loop(0, CHUNK // 32, group, None)

    win_load_start(0, 0)
    win_load_start(1, 1)
    win_load_wait(0, 0)
    gather_start(0, 0)
    gather_start(1, 1)

    def super_it(jj, carry):
        wn = jj // SPW + 1
        for par in range(2):
            @pl.when((jj % SPW == SPW - 1) & (wn < NWIN) & (wn % 2 == par))
            def _wait_next_window(par=par):
                win_load_wait(wn, par)

        for b in range(4):
            j = jj * 4 + b
            gather_wait(j, b)
            scale(j, b)
            scatter_start(j, b)
            jn = j + 2
            bn_ = (b + 2) % 4

            @pl.when(jn < NCHUNK)
            def _start_next():
                @pl.when(jn >= 4)
                def _free_buf():
                    scatter_wait(jn - 4, bn_)
                gather_start(jn, bn_)

        for par in range(2):
            @pl.when((jj % SPW == 0) & (jj >= SPW) & (jj <= SPW * (NWIN - 2))
                     & (wn % 2 == par))
            def _load_next_window(par=par):
                win_load_start(wn, par)
        return carry
    lax.fori_loop(0, NCHUNK // 4, super_it, None)
    for b in range(4):
        scatter_wait(NCHUNK - 4 + b, b)
    plsc.subcore_barrier()
    pltpu.sync_copy(acc.at[pl.ds(s * RPS, RPS)],
                    out_hbm.at[c, pl.ds(s * RPS, RPS)])


def _dinv_col(degp):
    deg = jnp.sum(degp, axis=0) + 1.0
    dinv = lax.rsqrt(jnp.maximum(deg, 1e-12))
    dinv = jnp.where(deg > 0, dinv, 0.0)
    return dinv[:N].reshape(N, 1)


def _tc_mm_body(x_ref, w1_ref, xw_ref):
    xw_ref[...] = jnp.dot(x_ref[...], w1_ref[...],
                          preferred_element_type=jnp.float32)


def _tc_scale_body(xw_ref, degp_ref, xs1_ref):
    xs1_ref[...] = xw_ref[...] * _dinv_col(degp_ref[...])


def _tc_mid_body(acc_ref, xs_ref, degp_ref, b_ref, g_ref, be_ref, w_ref, out_ref):
    dinv = _dinv_col(degp_ref[...])
    accs = acc_ref[0, :N, :] + acc_ref[1, :N, :]
    h = dinv * (accs + xs_ref[...]) + b_ref[...]
    mean = jnp.mean(h, axis=0, keepdims=True)
    var = jnp.mean(h * h, axis=0, keepdims=True) - mean * mean
    h = (h - mean) * lax.rsqrt(var + 1e-5) * g_ref[...] + be_ref[...]
    h = jnp.maximum(h, 0.0)
    out_ref[...] = jnp.dot(h, w_ref[...], preferred_element_type=jnp.float32) * dinv


def _tc_final_body(acc_ref, xs_ref, degp_ref, b_ref, g_ref, be_ref,
                   batch_ref, wl_ref, bl_ref, out_ref):
    dinv = _dinv_col(degp_ref[...])
    accs = acc_ref[0, :N, :] + acc_ref[1, :N, :]
    h = dinv * (accs + xs_ref[...]) + b_ref[...]
    mean = jnp.mean(h, axis=0, keepdims=True)
    var = jnp.mean(h * h, axis=0, keepdims=True) - mean * mean
    h = (h - mean) * lax.rsqrt(var + 1e-5) * g_ref[...] + be_ref[...]
    h = jnp.maximum(h, 0.0)
    gids = lax.broadcasted_iota(jnp.int32, (1, G), 1)
    onehot = (batch_ref[...] == gids).astype(jnp.float32)
    sums = lax.dot_general(onehot, h, (((0,), (0,)), ((), ())),
                           preferred_element_type=jnp.float32)
    cnt = jnp.sum(onehot, axis=0).reshape(G, 1)
    pooled = sums / jnp.maximum(cnt, 1.0)
    out_ref[...] = jnp.dot(pooled, wl_ref[...],
                           preferred_element_type=jnp.float32) + bl_ref[...]


def kernel(x, edge_index, edge_attr, batch, W1, b1, g1, be1,
           W2, b2, g2, be2, Wl, bl):
    pad = E_PAD - E
    pad_idx = (jnp.arange(pad, dtype=jnp.int32) % N)
    src_f = jnp.concatenate([edge_index[0], pad_idx])
    dst_f = jnp.concatenate([edge_index[1], pad_idx])
    ew_f = jnp.concatenate([edge_attr, jnp.zeros((pad,), jnp.float32)])
    src = src_f.reshape(NW, NCHUNK, CHUNK)
    dst = dst_f.reshape(NW, NCHUNK, CHUNK)
    ew = ew_f.reshape(NW, NCHUNK, CHUNK)

    degp = _make_deg_kernel()(dst_f.reshape(NW, DNC, DCH),
                              ew_f.reshape(NW, DNC, DCH))

    xw1 = pl.pallas_call(
        _tc_mm_body,
        out_shape=jax.ShapeDtypeStruct((N, D), jnp.float32),
    )(x, W1)
    xs1 = pl.pallas_call(
        _tc_scale_body,
        out_shape=jax.ShapeDtypeStruct((N, D), jnp.float32),
    )(xw1, degp)

    acc1 = _make_edge_kernel()(xs1, src, dst, ew)

    xs2 = pl.pallas_call(
        _tc_mid_body,
        out_shape=jax.ShapeDtypeStruct((N, D), jnp.float32),
    )(acc1, xs1, degp, b1.reshape(1, D), g1.reshape(1, D), be1.reshape(1, D), W2)

    acc2 = _make_edge_kernel()(xs2, src, dst, ew)

    out = pl.pallas_call(
        _tc_final_body,
        out_shape=jax.ShapeDtypeStruct((G, C), jnp.float32),
    )(acc2, xs2, degp, b2.reshape(1, D), g2.reshape(1, D), be2.reshape(1, D),
      batch.reshape(N, 1), Wl, bl.reshape(1, C))
    return out


# speedup vs baseline: 1.0135x; 1.0135x over previous
import functools

import jax
import jax.numpy as jnp
from jax import lax
from jax.experimental import pallas as pl
from jax.experimental.pallas import tpu as pltpu
from jax.experimental.pallas import tpu_sc as plsc

N = 10000
E = 320000
D = 128
G = 64
C = 16

NC = 2
NS = 16
NW = NC * NS
CHUNK = 64
NCHUNK = 160
W = 16
NWIN = NCHUNK // W
SPW = W // 4
EPW = CHUNK * NCHUNK
E_PAD = NW * EPW
N_PAD = 10240
RPS = N_PAD // NS

DCH = 128
DNC = EPW // DCH


@functools.lru_cache(maxsize=None)
def _make_deg_kernel():
    mesh = plsc.VectorSubcoreMesh(core_axis_name="c", subcore_axis_name="s")
    return functools.partial(
        pl.kernel, mesh=mesh,
        out_type=jax.ShapeDtypeStruct((NC, N_PAD), jnp.float32),
        scratch_types=[
            pltpu.VMEM((DNC, DCH), jnp.int32),
            pltpu.VMEM((DNC, DCH), jnp.float32),
            pltpu.VMEM((RPS,), jnp.float32),
            pltpu.SemaphoreType.DMA,
            pltpu.SemaphoreType.DMA,
            pltpu.SemaphoreType.DMA,
            pltpu.SemaphoreType.DMA,
            pltpu.VMEM_SHARED((N_PAD,), jnp.float32),
        ],
    )(_deg_body)


def _deg_body(dst_hbm, ew_hbm, out_hbm, dst_v, ew_v, zbuf, ds0, ds1, ds2, ds3, sdeg):
    c = lax.axis_index("c")
    s = lax.axis_index("s")
    wid = s * NC + c
    pltpu.sync_copy(dst_hbm.at[wid], dst_v)
    pltpu.sync_copy(ew_hbm.at[wid], ew_v)

    def zb(k, carry):
        zbuf[pl.ds(k * 16, 16)] = jnp.zeros((16,), jnp.float32)
        return carry
    lax.fori_loop(0, RPS // 16, zb, None)
    pltpu.sync_copy(zbuf, sdeg.at[pl.ds(s * RPS, RPS)])
    plsc.subcore_barrier()

    dsem = [ds0, ds1, ds2, ds3]

    def chunk4(jj, carry):
        for b in range(4):
            j = jj * 4 + b
            pltpu.async_copy(ew_v.at[j], sdeg.at[dst_v.at[j]], dsem[b], add=True)

            @pl.when(j >= 4)
            def _drain():
                pltpu.make_async_copy(
                    ew_v.at[j - 4], sdeg.at[dst_v.at[j - 4]], dsem[b]).wait()
        return carry
    lax.fori_loop(0, DNC // 4, chunk4, None)
    for b in range(4):
        pltpu.make_async_copy(
            ew_v.at[DNC - 4 + b], sdeg.at[dst_v.at[DNC - 4 + b]],
            dsem[b]).wait()
    plsc.subcore_barrier()
    pltpu.sync_copy(sdeg.at[pl.ds(s * RPS, RPS)], out_hbm.at[c, pl.ds(s * RPS, RPS)])


@functools.lru_cache(maxsize=None)
def _make_edge_kernel():
    mesh = plsc.VectorSubcoreMesh(core_axis_name="c", subcore_axis_name="s")
    return functools.partial(
        pl.kernel, mesh=mesh,
        out_type=jax.ShapeDtypeStruct((NC, N_PAD, D), jnp.float32),
        scratch_types=[
            pltpu.VMEM((2, W, CHUNK), jnp.int32),
            pltpu.VMEM((2, W, CHUNK), jnp.int32),
            pltpu.VMEM((2, W, CHUNK), jnp.float32),
            pltpu.VMEM((4, CHUNK, D), jnp.float32),
            pltpu.SemaphoreType.DMA,
            pltpu.SemaphoreType.DMA,
            pltpu.SemaphoreType.DMA,
            pltpu.SemaphoreType.DMA,
            pltpu.SemaphoreType.DMA,
            pltpu.SemaphoreType.DMA,
            pltpu.SemaphoreType.DMA,
            pltpu.SemaphoreType.DMA,
            pltpu.SemaphoreType.DMA,
            pltpu.SemaphoreType.DMA,
            pltpu.VMEM_SHARED((N_PAD, D), jnp.float32),
        ],
    )(_edge_body)


def _edge_body(xs_hbm, src_hbm, dst_hbm, ew_hbm, out_hbm,
               win_src, win_dst, win_ew, rows4,
               gs0, gs1, gs2, gs3, ss0, ss1, ss2, ss3, ws0, ws1, acc):
    c = lax.axis_index("c")
    s = lax.axis_index("s")
    wid = s * NC + c
    gsem = [gs0, gs1, gs2, gs3]
    ssem = [ss0, ss1, ss2, ss3]
    wsem = [ws0, ws1]

    def win_ops(w, wb):
        return [
            (src_hbm.at[wid, pl.ds(w * W, W)], win_src.at[wb]),
            (dst_hbm.at[wid, pl.ds(w * W, W)], win_dst.at[wb]),
            (ew_hbm.at[wid, pl.ds(w * W, W)], win_ew.at[wb]),
        ]

    def win_load_start(w, wb):
        for sr, ds_ in win_ops(w, wb):
            pltpu.async_copy(sr, ds_, wsem[wb])

    def win_load_wait(w, wb):
        for sr, ds_ in win_ops(w, wb):
            pltpu.make_async_copy(sr, ds_, wsem[wb]).wait()

    def zr(i, carry):
        for h in range(D // 16):
            rows4[0, i, pl.ds(h * 16, 16)] = jnp.zeros((16,), jnp.float32)
        return carry
    lax.fori_loop(0, CHUNK, zr, None)
    for t in range(RPS // CHUNK):
        pltpu.async_copy(rows4.at[0], acc.at[pl.ds(s * RPS + t * CHUNK, CHUNK)],
                         gs0)
    for t in range(RPS // CHUNK):
        pltpu.make_async_copy(rows4.at[0],
                              acc.at[pl.ds(s * RPS + t * CHUNK, CHUNK)],
                              gs0).wait()
    plsc.subcore_barrier()

    def gather_start(j, b):
        pltpu.async_copy(xs_hbm.at[win_src.at[(j // W) % 2, j % W]],
                         rows4.at[b], gsem[b])

    def gather_wait(j, b):
        pltpu.make_async_copy(xs_hbm.at[win_src.at[(j // W) % 2, j % W]],
                              rows4.at[b], gsem[b]).wait()

    def scatter_start(j, b):
        pltpu.async_copy(rows4.at[b], acc.at[win_dst.at[(j // W) % 2, j % W]],
                         ssem[b], add=True)

    def scatter_wait(j, b):
        pltpu.make_async_copy(rows4.at[b], acc.at[win_dst.at[(j // W) % 2, j % W]],
                              ssem[b]).wait()

    def scale(j, b):
        wb = (j // W) % 2
        row = j % W

        def group(g2, carry):
            for u in range(2):
                g = g2 * 2 + u
                ew16 = win_ew[wb, row, pl.ds(g * 16, 16)]
                for k in range(16):
                    ewb = jnp.full((16,), ew16[k])
                    i = g * 16 + k
                    for h in range(D // 16):
                        sl = rows4[b, i, pl.ds(h * 16, 16)]
                        rows4[b, i, pl.ds(h * 16, 16)] = sl * ewb
            return carry
        lax.fori_loop(0, CHUNK // 32, group, None)

    win_load_start(0, 0)
    win_load_start(1, 1)
    win_load_wait(0, 0)
    gather_start(0, 0)
    gather_start(1, 1)

    def super_it(jj, carry):
        wn = jj // SPW + 1
        for par in range(2):
            @pl.when((jj % SPW == SPW - 1) & (wn < NWIN) & (wn % 2 == par))
            def _wait_next_window(par=par):
                win_load_wait(wn, par)

        for b in range(4):
            j = jj * 4 + b
            gather_wait(j, b)
            scale(j, b)
            scatter_start(j, b)
            jn = j + 2
            bn_ = (b + 2) % 4

            @pl.when(jn < NCHUNK)
            def _start_next():
                @pl.when(jn >= 4)
                def _free_buf():
                    scatter_wait(jn - 4, bn_)
                gather_start(jn, bn_)

        for par in range(2):
            @pl.when((jj % SPW == 0) & (jj >= SPW) & (jj <= SPW * (NWIN - 2))
                     & (wn % 2 == par))
            def _load_next_window(par=par):
                win_load_start(wn, par)
        return carry
    lax.fori_loop(0, NCHUNK // 4, super_it, None)
    for b in range(4):
        scatter_wait(NCHUNK - 4 + b, b)
    plsc.subcore_barrier()
    pltpu.sync_copy(acc.at[pl.ds(s * RPS, RPS)],
                    out_hbm.at[c, pl.ds(s * RPS, RPS)])


def _dinv_col(degp):
    deg = jnp.sum(degp, axis=0) + 1.0
    dinv = lax.rsqrt(jnp.maximum(deg, 1e-12))
    dinv = jnp.where(deg > 0, dinv, 0.0)
    return dinv[:N].reshape(N, 1)


def _tc1_body(x_ref, w1_ref, degp_ref, xs1_ref):
    dinv = _dinv_col(degp_ref[...])
    xw = jnp.dot(x_ref[...], w1_ref[...], preferred_element_type=jnp.float32)
    xs1_ref[...] = xw * dinv


def _tc_mid_body(acc_ref, xs_ref, degp_ref, b_ref, g_ref, be_ref, w_ref, out_ref):
    dinv = _dinv_col(degp_ref[...])
    accs = acc_ref[0, :N, :] + acc_ref[1, :N, :]
    h = dinv * (accs + xs_ref[...]) + b_ref[...]
    mean = jnp.mean(h, axis=0, keepdims=True)
    var = jnp.mean(h * h, axis=0, keepdims=True) - mean * mean
    h = (h - mean) * lax.rsqrt(var + 1e-5) * g_ref[...] + be_ref[...]
    h = jnp.maximum(h, 0.0)
    out_ref[...] = jnp.dot(h, w_ref[...], preferred_element_type=jnp.float32) * dinv


def _tc_final_body(acc_ref, xs_ref, degp_ref, b_ref, g_ref, be_ref,
                   batch_ref, wl_ref, bl_ref, out_ref):
    dinv = _dinv_col(degp_ref[...])
    accs = acc_ref[0, :N, :] + acc_ref[1, :N, :]
    h = dinv * (accs + xs_ref[...]) + b_ref[...]
    mean = jnp.mean(h, axis=0, keepdims=True)
    var = jnp.mean(h * h, axis=0, keepdims=True) - mean * mean
    h = (h - mean) * lax.rsqrt(var + 1e-5) * g_ref[...] + be_ref[...]
    h = jnp.maximum(h, 0.0)
    gids = lax.broadcasted_iota(jnp.int32, (1, G), 1)
    onehot = (batch_ref[...] == gids).astype(jnp.float32)
    sums = lax.dot_general(onehot, h, (((0,), (0,)), ((), ())),
                           preferred_element_type=jnp.float32)
    cnt = jnp.sum(onehot, axis=0).reshape(G, 1)
    pooled = sums / jnp.maximum(cnt, 1.0)
    out_ref[...] = jnp.dot(pooled, wl_ref[...],
                           preferred_element_type=jnp.float32) + bl_ref[...]


def kernel(x, edge_index, edge_attr, batch, W1, b1, g1, be1,
           W2, b2, g2, be2, Wl, bl):
    pad = E_PAD - E
    pad_idx = (jnp.arange(pad, dtype=jnp.int32) % N)
    src_f = jnp.concatenate([edge_index[0], pad_idx])
    dst_f = jnp.concatenate([edge_index[1], pad_idx])
    ew_f = jnp.concatenate([edge_attr, jnp.zeros((pad,), jnp.float32)])
    src = src_f.reshape(NW, NCHUNK, CHUNK)
    dst = dst_f.reshape(NW, NCHUNK, CHUNK)
    ew = ew_f.reshape(NW, NCHUNK, CHUNK)

    degp = _make_deg_kernel()(dst_f.reshape(NW, DNC, DCH),
                              ew_f.reshape(NW, DNC, DCH))

    xs1 = pl.pallas_call(
        _tc1_body,
        out_shape=jax.ShapeDtypeStruct((N, D), jnp.float32),
    )(x, W1, degp)

    acc1 = _make_edge_kernel()(xs1, src, dst, ew)

    xs2 = pl.pallas_call(
        _tc_mid_body,
        out_shape=jax.ShapeDtypeStruct((N, D), jnp.float32),
    )(acc1, xs1, degp, b1.reshape(1, D), g1.reshape(1, D), be1.reshape(1, D), W2)

    acc2 = _make_edge_kernel()(xs2, src, dst, ew)

    out = pl.pallas_call(
        _tc_final_body,
        out_shape=jax.ShapeDtypeStruct((G, C), jnp.float32),
    )(acc2, xs2, degp, b2.reshape(1, D), g2.reshape(1, D), be2.reshape(1, D),
      batch.reshape(N, 1), Wl, bl.reshape(1, C))
    return out
